# speedup vs baseline: 2.4207x; 2.4207x over previous
import functools
import math

import jax
import jax.numpy as jnp
from jax import lax
from jax.experimental import pallas as pl
from jax.experimental.pallas import tpu as pltpu
from jax.experimental.pallas import tpu_sc as plsc

D_MODEL = 128
SCALE = math.sqrt(float(D_MODEL))


def kernel(x, table):
    B0, B1 = x.shape
    V, D = table.shape
    info = plsc.get_sparse_core_info()
    NC, NS, L = info.num_cores, info.num_subcores, info.num_lanes
    NW = NC * NS
    total = B0 * B1
    CH = 128
    NJ = total // (NW * CH)
    assert NJ * CH * NW == total and D % L == 0

    xr = x.reshape(NW, NJ, CH).astype(jnp.int32)
    mesh = plsc.VectorSubcoreMesh(core_axis_name="c", subcore_axis_name="s")

    @functools.partial(
        pl.kernel,
        mesh=mesh,
        out_type=jax.ShapeDtypeStruct((NW, NJ, CH, D), jnp.float32),
        scratch_types=[
            pltpu.VMEM((NJ, CH), jnp.int32),
            pltpu.VMEM((CH, D), jnp.float32),
            pltpu.SemaphoreType.DMA,
        ],
    )
    def emb_kernel(x_hbm, table_hbm, out_hbm, idx_v, rows_v, sem):
        c = lax.axis_index("c")
        s = lax.axis_index("s")
        wid = s * NC + c
        pltpu.sync_copy(x_hbm.at[wid], idx_v)

        def step(j, carry):
            pltpu.async_copy(table_hbm.at[idx_v.at[j]], rows_v, sem).wait()

            def scale_row(r, inner):
                for t in range(D // 16):
                    sl = pl.ds(t * 16, 16)
                    rows_v[r, sl] = rows_v[r, sl] * SCALE
                return inner

            lax.fori_loop(0, CH, scale_row, 0)
            pltpu.sync_copy(rows_v, out_hbm.at[wid, j])
            return carry

        lax.fori_loop(0, NJ, step, 0)

    out = emb_kernel(xr, table)
    return out.reshape(B0, B1, D)


# speedup vs baseline: 2.8292x; 1.1687x over previous
import functools
import math

import jax
import jax.numpy as jnp
from jax import lax
from jax.experimental import pallas as pl
from jax.experimental.pallas import tpu as pltpu
from jax.experimental.pallas import tpu_sc as plsc

D_MODEL = 128
SCALE = math.sqrt(float(D_MODEL))


def kernel(x, table):
    B0, B1 = x.shape
    V, D = table.shape
    info = plsc.get_sparse_core_info()
    NC, NS, L = info.num_cores, info.num_subcores, info.num_lanes
    NW = NC * NS
    total = B0 * B1
    CH = 128
    NJ = total // (NW * CH)
    assert NJ * CH * NW == total and D % L == 0 and NJ >= 4 and NJ % 2 == 0

    xr = x.reshape(NW, NJ, CH).astype(jnp.int32)
    mesh = plsc.VectorSubcoreMesh(core_axis_name="c", subcore_axis_name="s")

    @functools.partial(
        pl.kernel,
        mesh=mesh,
        out_type=jax.ShapeDtypeStruct((NW, NJ, CH, D), jnp.float32),
        scratch_types=[
            pltpu.VMEM((NJ, CH), jnp.int32),
            pltpu.VMEM((2, CH, D), jnp.float32),
            pltpu.VMEM((2, CH, D), jnp.float32),
            pltpu.SemaphoreType.DMA,
            pltpu.SemaphoreType.DMA,
            pltpu.SemaphoreType.DMA,
            pltpu.SemaphoreType.DMA,
        ],
    )
    def emb_kernel(x_hbm, table_hbm, out_hbm, idx_v, rows_in, rows_out,
                   sem_g0, sem_g1, sem_s0, sem_s1):
        c = lax.axis_index("c")
        s = lax.axis_index("s")
        wid = s * NC + c
        pltpu.sync_copy(x_hbm.at[wid], idx_v)
        sem_g = (sem_g0, sem_g1)
        sem_s = (sem_s0, sem_s1)

        def g_copy(jj, b):
            return pltpu.make_async_copy(
                table_hbm.at[idx_v.at[jj]], rows_in.at[b], sem_g[b])

        def s_copy(jj, b):
            return pltpu.make_async_copy(
                rows_out.at[b], out_hbm.at[wid, jj], sem_s[b])

        def scale(b):
            def row(i, carry):
                r = i * 2
                for rr in range(2):
                    for t in range(D // 16):
                        sl = pl.ds(t * 16, 16)
                        rows_out[b, r + rr, sl] = rows_in[b, r + rr, sl] * SCALE
                return carry
            lax.fori_loop(0, CH // 2, row, 0)

        g_copy(0, 0).start()
        g_copy(0, 0).wait()
        g_copy(1, 1).start()
        scale(0)
        s_copy(0, 0).start()
        g_copy(1, 1).wait()
        g_copy(2, 0).start()
        scale(1)
        s_copy(1, 1).start()

        def pair(p, carry):
            jj0 = 2 * p + 2
            for b in range(2):
                jj = jj0 + b
                g_copy(jj, b).wait()
                s_copy(jj - 2, b).wait()
                g_copy(jj + 1, 1 - b).start()
                scale(b)
                s_copy(jj, b).start()
            return carry
        lax.fori_loop(0, (NJ - 4) // 2, pair, 0)

        g_copy(NJ - 2, 0).wait()
        s_copy(NJ - 4, 0).wait()
        g_copy(NJ - 1, 1).start()
        scale(0)
        s_copy(NJ - 2, 0).start()
        g_copy(NJ - 1, 1).wait()
        s_copy(NJ - 3, 1).wait()
        scale(1)
        s_copy(NJ - 1, 1).start()
        s_copy(NJ - 2, 0).wait()
        s_copy(NJ - 1, 1).wait()

    out = emb_kernel(xr, table)
    return out.reshape(B0, B1, D)
